# baseline (speedup 1.0000x reference)
import functools

import jax
import jax.numpy as jnp
from jax import lax
from jax.experimental import pallas as pl
from jax.experimental.pallas import tpu as pltpu
from jax.experimental.pallas import tpu_sc as plsc

VOCAB = 1000000
D = 64
BATCH = 4096
HIST = 50
B_TOTAL = BATCH * HIST
NC = 2
NS = 16
NW = NC * NS
B_PER_W = B_TOTAL // NW
CHUNK = 128
NCHUNKS = B_PER_W // CHUNK

_MESH = plsc.VectorSubcoreMesh(core_axis_name="c", subcore_axis_name="s")


@functools.partial(
    pl.kernel,
    out_type=jax.ShapeDtypeStruct((B_TOTAL, D), jnp.float32),
    mesh=_MESH,
    scratch_types=[
        pltpu.VMEM((NCHUNKS, CHUNK), jnp.int32),
        pltpu.VMEM((CHUNK, D), jnp.float32),
        pltpu.SemaphoreType.DMA,
    ],
    compiler_params=pltpu.CompilerParams(use_tc_tiling_on_sc=False),
)
def _sc_gather(idx_hbm, table_hbm, out_hbm, idx_v, buf, gsem):
    wid = lax.axis_index("s") * NC + lax.axis_index("c")
    base = wid * B_PER_W
    pltpu.sync_copy(idx_hbm.at[wid], idx_v)

    def chunk_body(c, carry):
        pltpu.async_copy(table_hbm.at[idx_v.at[c]], buf, gsem).wait()
        pltpu.sync_copy(buf, out_hbm.at[pl.ds(base + c * CHUNK, CHUNK)])
        return carry

    lax.fori_loop(0, NCHUNKS, chunk_body, 0)


def kernel(x, embedding_table):
    idx = x.reshape(-1).astype(jnp.int32).reshape(NW, NCHUNKS, CHUNK)
    out = _sc_gather(idx, embedding_table)
    return out.reshape(BATCH, HIST, D)


# speedup vs baseline: 1.0236x; 1.0236x over previous
import functools

import jax
import jax.numpy as jnp
from jax import lax
from jax.experimental import pallas as pl
from jax.experimental.pallas import tpu as pltpu
from jax.experimental.pallas import tpu_sc as plsc

VOCAB = 1000000
D = 64
BATCH = 4096
HIST = 50
NC = 2
NS = 16
NW = NC * NS
BATCH_PER_W = BATCH // NW
CHUNK_B = 2
CHUNK = CHUNK_B * HIST
NCHUNKS = BATCH_PER_W // CHUNK_B

_MESH = plsc.VectorSubcoreMesh(core_axis_name="c", subcore_axis_name="s")


@functools.partial(
    pl.kernel,
    out_type=jax.ShapeDtypeStruct((BATCH, HIST, D), jnp.float32),
    mesh=_MESH,
    scratch_types=[
        pltpu.VMEM((NCHUNKS, CHUNK), jnp.int32),
        pltpu.VMEM((CHUNK, D), jnp.float32),
        pltpu.VMEM((CHUNK, D), jnp.float32),
        pltpu.SemaphoreType.DMA,
        pltpu.SemaphoreType.DMA,
    ],
    compiler_params=pltpu.CompilerParams(use_tc_tiling_on_sc=False),
)
def _sc_gather(idx_hbm, table_hbm, out_hbm, idx_v, buf0, buf1, sem0, sem1):
    wid = lax.axis_index("s") * NC + lax.axis_index("c")
    base_b = wid * BATCH_PER_W
    pltpu.sync_copy(idx_hbm.at[wid], idx_v)

    def start(c, buf, sem):
        pltpu.async_copy(table_hbm.at[idx_v.at[c]], buf, sem)

    def finish(c, buf, sem):
        pltpu.make_async_copy(
            table_hbm.at[idx_v.at[0]], buf, sem).wait()
        b = base_b + c * CHUNK_B
        pltpu.sync_copy(buf.at[pl.ds(0, HIST)], out_hbm.at[b])
        pltpu.sync_copy(buf.at[pl.ds(HIST, HIST)], out_hbm.at[b + 1])

    start(0, buf0, sem0)

    def pair_body(cc, carry):
        c0 = 2 * cc
        start(c0 + 1, buf1, sem1)
        finish(c0, buf0, sem0)

        @pl.when(c0 + 2 < NCHUNKS)
        def _():
            start(c0 + 2, buf0, sem0)

        finish(c0 + 1, buf1, sem1)
        return carry

    lax.fori_loop(0, NCHUNKS // 2, pair_body, 0)


def kernel(x, embedding_table):
    idx = x.reshape(-1).astype(jnp.int32).reshape(NW, NCHUNKS, CHUNK)
    return _sc_gather(idx, embedding_table)
